# baseline (speedup 1.0000x reference)
import functools

import jax
import jax.numpy as jnp
from jax import lax
from jax.experimental import pallas as pl
from jax.experimental.pallas import tpu as pltpu
from jax.experimental.pallas import tpu_sc as plsc

N = 10000
E = 160000
D = 256
H = 128

N_PAD = 10240
E_CHUNKS = 1280
E_PAD = E_CHUNKS * 128
CPT = E_CHUNKS // 16
CPT_DEG = E_CHUNKS // 32
RPT = N_PAD // 16
NBUF = 2
CPT2 = CPT // 2

BN = 1024


def _deg_body(colb, out0, out1, col_v, ones_v, zbuf, deg_sh):
  c = lax.axis_index("c")
  s = lax.axis_index("s")
  for k in range(8):
    ones_v[pl.ds(k * 16, 16)] = jnp.ones((16,), jnp.float32)
  for k in range(RPT // 16):
    zbuf[pl.ds(k * 16, 16)] = jnp.zeros((16,), jnp.float32)
  pltpu.sync_copy(zbuf, deg_sh.at[pl.ds(s * RPT, RPT)])
  base = (c * 16 + s) * CPT_DEG
  pltpu.sync_copy(colb.at[pl.ds(base, CPT_DEG)], col_v)
  plsc.subcore_barrier()

  def body(j, carry):
    pltpu.sync_copy(ones_v, deg_sh.at[col_v.at[j]], add=True)
    return carry

  lax.fori_loop(0, CPT_DEG, body, 0)
  plsc.subcore_barrier()

  @pl.when(c == 0)
  def _():
    pltpu.sync_copy(deg_sh.at[pl.ds(s * RPT, RPT)], out0.at[pl.ds(s * RPT, RPT)])

  @pl.when(c == 1)
  def _():
    pltpu.sync_copy(deg_sh.at[pl.ds(s * RPT, RPT)], out1.at[pl.ds(s * RPT, RPT)])


_deg_kernel = functools.partial(
    pl.kernel,
    out_type=[
        jax.ShapeDtypeStruct((N_PAD,), jnp.float32),
        jax.ShapeDtypeStruct((N_PAD,), jnp.float32),
    ],
    mesh=plsc.VectorSubcoreMesh(core_axis_name="c", subcore_axis_name="s"),
    scratch_types=[
        pltpu.VMEM((CPT_DEG, 128), jnp.int32),
        pltpu.VMEM((128,), jnp.float32),
        pltpu.VMEM((RPT,), jnp.float32),
        pltpu.VMEM_SHARED((N_PAD,), jnp.float32),
    ],
)(_deg_body)


def _agg_body(xs0, xs1, colb, rowb, zeros2d, out,
              col_v, row_v, gb0, gb1, sm0, sm1, sm2, sm3, agg_sh):
  gbufs = (gb0, gb1)
  gsems = (sm0, sm1)
  ssems = (sm2, sm3)
  c = lax.axis_index("c")
  s = lax.axis_index("s")

  def run(table, half):
    def gather(j, k):
      return pltpu.async_copy(table.at[col_v.at[j]], gbufs[k], gsems[k])

    def gwait(j, k):
      pltpu.make_async_copy(table.at[col_v.at[j]], gbufs[k], gsems[k]).wait()

    def scatter(j, k):
      pltpu.async_copy(gbufs[k], agg_sh.at[row_v.at[j]], ssems[k], add=True)

    def swait(j, k):
      pltpu.make_async_copy(gbufs[k], agg_sh.at[row_v.at[j]], ssems[k]).wait()

    for phase in range(2):
      base = s * CPT + phase * CPT2
      pltpu.sync_copy(colb.at[pl.ds(base, CPT2)], col_v)
      pltpu.sync_copy(rowb.at[pl.ds(base, CPT2)], row_v)
      for k in range(NBUF):
        gather(k, k)
      if phase == 0:
        pltpu.sync_copy(zeros2d.at[pl.ds(s * RPT, RPT)],
                        agg_sh.at[pl.ds(s * RPT, RPT)])
        plsc.subcore_barrier()

      def body2(i, carry):
        j = i * NBUF
        for k in range(NBUF):
          gwait(j + k, k)
          scatter(j + k, k)
        for k in range(NBUF):
          swait(j + k, k)
          gather(j + NBUF + k, k)
        return carry

      lax.fori_loop(0, CPT2 // NBUF - 1, body2, 0)
      jlast = CPT2 - NBUF
      for k in range(NBUF):
        gwait(jlast + k, k)
        scatter(jlast + k, k)
      for k in range(NBUF):
        swait(jlast + k, k)

    plsc.subcore_barrier()
    pltpu.sync_copy(agg_sh.at[pl.ds(s * RPT, RPT)],
                    out.at[half].at[pl.ds(s * RPT, RPT)])

  @pl.when(c == 0)
  def _():
    run(xs0, 0)

  @pl.when(c == 1)
  def _():
    run(xs1, 1)


_agg_kernel = functools.partial(
    pl.kernel,
    out_type=jax.ShapeDtypeStruct((2, N_PAD, H), jnp.float32),
    mesh=plsc.VectorSubcoreMesh(core_axis_name="c", subcore_axis_name="s"),
    scratch_types=[
        pltpu.VMEM((CPT2, 128), jnp.int32),
        pltpu.VMEM((CPT2, 128), jnp.int32),
        pltpu.VMEM((128, H), jnp.float32),
        pltpu.VMEM((128, H), jnp.float32),
        pltpu.SemaphoreType.DMA,
        pltpu.SemaphoreType.DMA,
        pltpu.SemaphoreType.DMA,
        pltpu.SemaphoreType.DMA,
        pltpu.VMEM_SHARED((N_PAD, H), jnp.float32),
    ],
)(_agg_body)


def _scale_body(d0_ref, d1_ref, x_ref, xs0_ref, xs1_ref):
  deg = d0_ref[...] + d1_ref[...] + 1.0
  dinv = lax.rsqrt(deg)
  xs = x_ref[...] * dinv
  xs0_ref[...] = xs[:, :H]
  xs1_ref[...] = xs[:, H:]


def _scale(d0, d1, x):
  grid = (N_PAD // BN,)
  return pl.pallas_call(
      _scale_body,
      grid=grid,
      in_specs=[
          pl.BlockSpec((BN, 1), lambda i: (i, 0)),
          pl.BlockSpec((BN, 1), lambda i: (i, 0)),
          pl.BlockSpec((BN, D), lambda i: (i, 0)),
      ],
      out_specs=[
          pl.BlockSpec((BN, H), lambda i: (i, 0)),
          pl.BlockSpec((BN, H), lambda i: (i, 0)),
      ],
      out_shape=[
          jax.ShapeDtypeStruct((N_PAD, H), jnp.float32),
          jax.ShapeDtypeStruct((N_PAD, H), jnp.float32),
      ],
  )(d0, d1, x)


def _out_body(d0_ref, d1_ref, x_ref, sr0_ref, sr1_ref, wt_ref, b_ref, o_ref):
  deg = d0_ref[...] + d1_ref[...] + 1.0
  dinv = lax.rsqrt(deg)
  sr = jnp.concatenate([sr0_ref[...], sr1_ref[...]], axis=1)
  scaled = (sr + dinv * x_ref[...]) * dinv
  o_ref[...] = lax.dot_general(
      scaled, wt_ref[...], (((1,), (1,)), ((), ())),
      preferred_element_type=jnp.float32,
  ) + b_ref[...]


def _final(d0, d1, x, sr0, sr1, wt, b2):
  grid = (N_PAD // BN,)
  return pl.pallas_call(
      _out_body,
      grid=grid,
      in_specs=[
          pl.BlockSpec((BN, 1), lambda i: (i, 0)),
          pl.BlockSpec((BN, 1), lambda i: (i, 0)),
          pl.BlockSpec((BN, D), lambda i: (i, 0)),
          pl.BlockSpec((BN, H), lambda i: (i, 0)),
          pl.BlockSpec((BN, H), lambda i: (i, 0)),
          pl.BlockSpec((D, D), lambda i: (0, 0)),
          pl.BlockSpec((1, D), lambda i: (0, 0)),
      ],
      out_specs=pl.BlockSpec((BN, D), lambda i: (i, 0)),
      out_shape=jax.ShapeDtypeStruct((N, D), jnp.float32),
  )(d0, d1, x, sr0, sr1, wt, b2)


@jax.jit
def kernel(x, edge_index, W, b):
  ei = edge_index.astype(jnp.int32)
  pad = N + jnp.arange(E_PAD - E, dtype=jnp.int32) % (N_PAD - N)
  rowb = jnp.concatenate([ei[0], pad]).reshape(E_CHUNKS, 128)
  colb = jnp.concatenate([ei[1], pad]).reshape(E_CHUNKS, 128)
  zeros2d = jnp.zeros((N_PAD, H), jnp.float32)

  deg0, deg1 = _deg_kernel(colb)
  d0 = deg0.reshape(N_PAD, 1)
  d1 = deg1.reshape(N_PAD, 1)
  xs0, xs1 = _scale(d0, d1, x)
  sr = _agg_kernel(xs0, xs1, colb, rowb, zeros2d)
  out = _final(d0, d1, x, sr[0], sr[1], W, b.reshape(1, D))
  return out


# speedup vs baseline: 1.1971x; 1.1971x over previous
import functools

import jax
import jax.numpy as jnp
from jax import lax
from jax.experimental import pallas as pl
from jax.experimental.pallas import tpu as pltpu
from jax.experimental.pallas import tpu_sc as plsc

N = 10000
E = 160000
D = 256
H = 128

N_PAD = 10240
E_CHUNKS = 1280
E_PAD = E_CHUNKS * 128
CPT = E_CHUNKS // 16
CPT_DEG = E_CHUNKS // 32
RPT = N_PAD // 16
NBUF = 2
CPT2 = CPT // 2

BN = 1024


def _deg_body(colb, out0, out1, col_v, ones_v, zbuf, deg_sh):
  c = lax.axis_index("c")
  s = lax.axis_index("s")
  for k in range(8):
    ones_v[pl.ds(k * 16, 16)] = jnp.ones((16,), jnp.float32)
  for k in range(RPT // 16):
    zbuf[pl.ds(k * 16, 16)] = jnp.zeros((16,), jnp.float32)
  pltpu.sync_copy(zbuf, deg_sh.at[pl.ds(s * RPT, RPT)])
  base = (c * 16 + s) * CPT_DEG
  pltpu.sync_copy(colb.at[pl.ds(base, CPT_DEG)], col_v)
  plsc.subcore_barrier()

  def body(j, carry):
    pltpu.sync_copy(ones_v, deg_sh.at[col_v.at[j]], add=True)
    return carry

  lax.fori_loop(0, CPT_DEG, body, 0)
  plsc.subcore_barrier()

  @pl.when(c == 0)
  def _():
    pltpu.sync_copy(deg_sh.at[pl.ds(s * RPT, RPT)], out0.at[pl.ds(s * RPT, RPT)])

  @pl.when(c == 1)
  def _():
    pltpu.sync_copy(deg_sh.at[pl.ds(s * RPT, RPT)], out1.at[pl.ds(s * RPT, RPT)])


_deg_kernel = functools.partial(
    pl.kernel,
    out_type=[
        jax.ShapeDtypeStruct((N_PAD,), jnp.float32),
        jax.ShapeDtypeStruct((N_PAD,), jnp.float32),
    ],
    mesh=plsc.VectorSubcoreMesh(core_axis_name="c", subcore_axis_name="s"),
    scratch_types=[
        pltpu.VMEM((CPT_DEG, 128), jnp.int32),
        pltpu.VMEM((128,), jnp.float32),
        pltpu.VMEM((RPT,), jnp.float32),
        pltpu.VMEM_SHARED((N_PAD,), jnp.float32),
    ],
)(_deg_body)


def _agg_body(xs0, xs1, colb, rowb, zeros2d, out,
              col_v, row_v, gb0, gb1, sm0, sm1, agg_sh):
  gbufs = (gb0, gb1)
  sems = (sm0, sm1)
  c = lax.axis_index("c")
  s = lax.axis_index("s")

  def run(table, half):
    def gather(j, k):
      return pltpu.async_copy(table.at[col_v.at[j]], gbufs[k], sems[k])

    def scatter(j, k):
      pltpu.sync_copy(gbufs[k], agg_sh.at[row_v.at[j]], add=True)

    for phase in range(2):
      base = s * CPT + phase * CPT2
      pltpu.sync_copy(colb.at[pl.ds(base, CPT2)], col_v)
      pltpu.sync_copy(rowb.at[pl.ds(base, CPT2)], row_v)
      for k in range(NBUF):
        gather(k, k)
      if phase == 0:
        pltpu.sync_copy(zeros2d.at[pl.ds(s * RPT, RPT)],
                        agg_sh.at[pl.ds(s * RPT, RPT)])
        plsc.subcore_barrier()

      def body2(i, carry):
        j = i * NBUF
        for k in range(NBUF):
          pltpu.make_async_copy(table.at[col_v.at[j + k]], gbufs[k], sems[k]).wait()
          scatter(j + k, k)
          gather(j + NBUF + k, k)
        return carry

      lax.fori_loop(0, CPT2 // NBUF - 1, body2, 0)
      jlast = CPT2 - NBUF
      for k in range(NBUF):
        pltpu.make_async_copy(table.at[col_v.at[jlast + k]], gbufs[k], sems[k]).wait()
        scatter(jlast + k, k)

    plsc.subcore_barrier()
    pltpu.sync_copy(agg_sh.at[pl.ds(s * RPT, RPT)],
                    out.at[half].at[pl.ds(s * RPT, RPT)])

  @pl.when(c == 0)
  def _():
    run(xs0, 0)

  @pl.when(c == 1)
  def _():
    run(xs1, 1)


_agg_kernel = functools.partial(
    pl.kernel,
    out_type=jax.ShapeDtypeStruct((2, N_PAD, H), jnp.float32),
    mesh=plsc.VectorSubcoreMesh(core_axis_name="c", subcore_axis_name="s"),
    scratch_types=[
        pltpu.VMEM((CPT2, 128), jnp.int32),
        pltpu.VMEM((CPT2, 128), jnp.int32),
        pltpu.VMEM((128, H), jnp.float32),
        pltpu.VMEM((128, H), jnp.float32),
        pltpu.SemaphoreType.DMA,
        pltpu.SemaphoreType.DMA,
        pltpu.VMEM_SHARED((N_PAD, H), jnp.float32),
    ],
)(_agg_body)


def _scale_body(d0_ref, d1_ref, x_ref, xs0_ref, xs1_ref):
  deg = d0_ref[...] + d1_ref[...] + 1.0
  dinv = lax.rsqrt(deg)
  xs = x_ref[...] * dinv
  xs0_ref[...] = xs[:, :H]
  xs1_ref[...] = xs[:, H:]


def _scale(d0, d1, x):
  grid = (N_PAD // BN,)
  return pl.pallas_call(
      _scale_body,
      grid=grid,
      in_specs=[
          pl.BlockSpec((BN, 1), lambda i: (i, 0)),
          pl.BlockSpec((BN, 1), lambda i: (i, 0)),
          pl.BlockSpec((BN, D), lambda i: (i, 0)),
      ],
      out_specs=[
          pl.BlockSpec((BN, H), lambda i: (i, 0)),
          pl.BlockSpec((BN, H), lambda i: (i, 0)),
      ],
      out_shape=[
          jax.ShapeDtypeStruct((N_PAD, H), jnp.float32),
          jax.ShapeDtypeStruct((N_PAD, H), jnp.float32),
      ],
  )(d0, d1, x)


def _out_body(d0_ref, d1_ref, x_ref, sr0_ref, sr1_ref, wt_ref, b_ref, o_ref):
  deg = d0_ref[...] + d1_ref[...] + 1.0
  dinv = lax.rsqrt(deg)
  sr = jnp.concatenate([sr0_ref[...], sr1_ref[...]], axis=1)
  scaled = (sr + dinv * x_ref[...]) * dinv
  o_ref[...] = lax.dot_general(
      scaled, wt_ref[...], (((1,), (1,)), ((), ())),
      preferred_element_type=jnp.float32,
  ) + b_ref[...]


def _final(d0, d1, x, sr0, sr1, wt, b2):
  grid = (N_PAD // BN,)
  return pl.pallas_call(
      _out_body,
      grid=grid,
      in_specs=[
          pl.BlockSpec((BN, 1), lambda i: (i, 0)),
          pl.BlockSpec((BN, 1), lambda i: (i, 0)),
          pl.BlockSpec((BN, D), lambda i: (i, 0)),
          pl.BlockSpec((BN, H), lambda i: (i, 0)),
          pl.BlockSpec((BN, H), lambda i: (i, 0)),
          pl.BlockSpec((D, D), lambda i: (0, 0)),
          pl.BlockSpec((1, D), lambda i: (0, 0)),
      ],
      out_specs=pl.BlockSpec((BN, D), lambda i: (i, 0)),
      out_shape=jax.ShapeDtypeStruct((N, D), jnp.float32),
  )(d0, d1, x, sr0, sr1, wt, b2)


@jax.jit
def kernel(x, edge_index, W, b):
  ei = edge_index.astype(jnp.int32)
  pad = N + jnp.arange(E_PAD - E, dtype=jnp.int32) % (N_PAD - N)
  rowb = jnp.concatenate([ei[0], pad]).reshape(E_CHUNKS, 128)
  colb = jnp.concatenate([ei[1], pad]).reshape(E_CHUNKS, 128)
  zeros2d = jnp.zeros((N_PAD, H), jnp.float32)

  deg0, deg1 = _deg_kernel(colb)
  d0 = deg0.reshape(N_PAD, 1)
  d1 = deg1.reshape(N_PAD, 1)
  xs0, xs1 = _scale(d0, d1, x)
  sr = _agg_kernel(xs0, xs1, colb, rowb, zeros2d)
  out = _final(d0, d1, x, sr[0], sr[1], W, b.reshape(1, D))
  return out
